# baseline (speedup 1.0000x reference)
import functools

import jax
import jax.numpy as jnp
from jax import lax
from jax.experimental import pallas as pl
from jax.experimental.pallas import tpu as pltpu
from jax.experimental.pallas import tpu_sc as plsc

N = 10000
E = 320000
D = 128
H = 16
O = 16
G = 128

NC = 2
NS = 16
NW = NC * NS
EP = E // NW
CH = 1000
NCH = EP // CH
NZ = N // NS
NP = N // 8


def _sc_scatter_body(with_deg, *refs):
    if with_deg:
        (p_hbm, ei_hbm, agg0, agg1, deg0, deg1,
         idx_all, dst_all, rows0, rows1, ones_v, agg_sh, deg_sh,
         isem, gsem, asem, dsem) = refs
    else:
        (p_hbm, ei_hbm, agg0, agg1,
         idx_all, dst_all, rows0, rows1, agg_sh,
         isem, gsem, asem) = refs

    c = lax.axis_index("c")
    s = lax.axis_index("s")
    wid = s * NC + c
    base = wid * EP

    idesc = []
    for i in range(NCH):
        off = base + i * CH
        idesc.append(pltpu.async_copy(
            ei_hbm.at[pl.ds(off, CH)], idx_all.at[i], isem))
        idesc.append(pltpu.async_copy(
            ei_hbm.at[pl.ds(E + off, CH)], dst_all.at[i], isem))

    def _zrow(i, _):
        rows0[i, :] = jnp.zeros((16,), jnp.float32)
        return 0
    lax.fori_loop(0, NZ, _zrow, 0)
    if with_deg:
        def _orow(i, _):
            ones_v[i, :] = jnp.ones((16,), jnp.float32)
            return 0
        lax.fori_loop(0, CH, _orow, 0)

    pltpu.sync_copy(rows0.at[pl.ds(0, NZ)], agg_sh.at[pl.ds(s * NZ, NZ)])
    if with_deg:
        pltpu.sync_copy(rows0.at[pl.ds(0, NZ)], deg_sh.at[pl.ds(s * NZ, NZ)])
    plsc.subcore_barrier()
    for d in idesc:
        d.wait()

    rows = (rows0, rows1)
    g = [None] * NCH
    a = [None] * NCH
    dd = []
    g[0] = pltpu.async_copy(p_hbm.at[idx_all.at[0]], rows[0], gsem)
    for i in range(NCH):
        if i + 1 < NCH:
            if i >= 1:
                a[i - 1].wait()
            g[i + 1] = pltpu.async_copy(
                p_hbm.at[idx_all.at[i + 1]], rows[(i + 1) % 2], gsem)
        g[i].wait()
        a[i] = pltpu.async_copy(
            rows[i % 2], agg_sh.at[dst_all.at[i]], asem, add=True)
        if with_deg:
            dd.append(pltpu.async_copy(
                ones_v, deg_sh.at[dst_all.at[i]], dsem, add=True))
    for i in range(max(0, NCH - 2), NCH):
        a[i].wait()
    for d in dd:
        d.wait()
    plsc.subcore_barrier()

    rb = s * NZ

    @pl.when(c == 0)
    def _():
        pltpu.sync_copy(agg_sh.at[pl.ds(rb, NZ)], agg0.at[pl.ds(rb, NZ)])
        if with_deg:
            pltpu.sync_copy(deg_sh.at[pl.ds(rb, NZ)], deg0.at[pl.ds(rb, NZ)])

    @pl.when(c == 1)
    def _():
        pltpu.sync_copy(agg_sh.at[pl.ds(rb, NZ)], agg1.at[pl.ds(rb, NZ)])
        if with_deg:
            pltpu.sync_copy(deg_sh.at[pl.ds(rb, NZ)], deg1.at[pl.ds(rb, NZ)])


def _make_sc_scatter(with_deg):
    n_out = 4 if with_deg else 2
    scratch = [
        pltpu.VMEM((NCH, CH), jnp.int32),
        pltpu.VMEM((NCH, CH), jnp.int32),
        pltpu.VMEM((CH, 16), jnp.float32),
        pltpu.VMEM((CH, 16), jnp.float32),
    ]
    if with_deg:
        scratch.append(pltpu.VMEM((CH, 16), jnp.float32))
    scratch.append(pltpu.VMEM_SHARED((N, 16), jnp.float32))
    if with_deg:
        scratch.append(pltpu.VMEM_SHARED((N, 16), jnp.float32))
    scratch.extend([pltpu.SemaphoreType.DMA] * (4 if with_deg else 3))

    mesh = plsc.VectorSubcoreMesh(core_axis_name="c", subcore_axis_name="s")
    return functools.partial(
        pl.kernel,
        mesh=mesh,
        out_type=[jax.ShapeDtypeStruct((N, 16), jnp.float32)] * n_out,
        scratch_types=scratch,
        compiler_params=pltpu.CompilerParams(use_tc_tiling_on_sc=False),
    )(functools.partial(_sc_scatter_body, with_deg))


def _full_spec(shape):
    return pl.BlockSpec(shape, lambda: tuple(0 for _ in shape))


def _dot(a, b):
    return lax.dot_general(a, b, (((1,), (0,)), ((), ())),
                           preferred_element_type=jnp.float32)


def _dot_t(a, b):
    return lax.dot_general(a, b, (((1,), (1,)), ((), ())),
                           preferred_element_type=jnp.float32)


def _blockdiag_matmul(xin, w_t, din):
    outs = [_dot_t(xin[:, din * i:din * (i + 1)], w_t) for i in range(8)]
    return jnp.concatenate(outs, axis=1)


def _projp_body(x3_ref, wl_ref, wr_ref, p_ref, r_ref):
    wl, wr = wl_ref[...], wr_ref[...]
    ps, rs = [], []
    for i in range(8):
        xi = x3_ref[:, i, :]
        ps.append(_dot_t(xi, wl))
        rs.append(_dot_t(xi, wr))
    p_ref[...] = jnp.concatenate(ps, axis=1)
    r_ref[...] = jnp.concatenate(rs, axis=1)


def _midp_body(a0_ref, a1_ref, d0_ref, d1_ref, r1_ref, b1_ref,
               w2l_ref, w2r_ref, b2_ref, p2_ref, r2b_ref):
    deg = jnp.maximum(d0_ref[...] + d1_ref[...], 1.0)
    h = jnp.maximum((a0_ref[...] + a1_ref[...]) / deg
                    + b1_ref[...] + r1_ref[...], 0.0)
    p2_ref[...] = _blockdiag_matmul(h, w2l_ref[...], H)
    r2b_ref[...] = _blockdiag_matmul(h, w2r_ref[...], H) + b2_ref[...]


def _finalp_body(g0_ref, g1_ref, d0_ref, d1_ref, r2b_ref, bt_ref, out_ref):
    deg = jnp.maximum(d0_ref[...] + d1_ref[...], 1.0)
    nodep = (g0_ref[...] + g1_ref[...]) / deg + r2b_ref[...]
    acc = jnp.zeros((G, O), jnp.float32)
    cnt = jnp.zeros((G, 1), jnp.float32)
    for i in range(8):
        bv = bt_ref[i:i + 1, :]
        oh = (lax.broadcasted_iota(jnp.int32, (G, NP), 0)
              == jnp.broadcast_to(bv, (G, NP))).astype(jnp.float32)
        acc += _dot(oh, nodep[:, 16 * i:16 * i + 16])
        cnt += jnp.sum(oh, axis=1, keepdims=True)
    out_ref[...] = acc / jnp.maximum(cnt, 1.0)


def _projp(xp, wl, wr):
    return pl.pallas_call(
        _projp_body,
        in_specs=[_full_spec((NP, 8, D)), _full_spec((H, D)),
                  _full_spec((H, D))],
        out_specs=[_full_spec((NP, 128))] * 2,
        out_shape=[jax.ShapeDtypeStruct((NP, 128), jnp.float32)] * 2,
    )(xp, wl, wr)


def _midp(a0p, a1p, d0p, d1p, r1p, b1p, w2lbd, w2rbd, b2p):
    return pl.pallas_call(
        _midp_body,
        in_specs=[_full_spec((NP, 128))] * 5
        + [_full_spec((1, 128)), _full_spec((O, H)),
           _full_spec((O, H)), _full_spec((1, 128))],
        out_specs=[_full_spec((NP, 128))] * 2,
        out_shape=[jax.ShapeDtypeStruct((NP, 128), jnp.float32)] * 2,
    )(a0p, a1p, d0p, d1p, r1p, b1p, w2lbd, w2rbd, b2p)


def _finalp(g0p, g1p, d0p, d1p, r2bp, bt):
    return pl.pallas_call(
        _finalp_body,
        in_specs=[_full_spec((NP, 128))] * 5 + [_full_spec((8, NP))],
        out_specs=_full_spec((G, O)),
        out_shape=jax.ShapeDtypeStruct((G, O), jnp.float32),
    )(g0p, g1p, d0p, d1p, r2bp, bt)


_sc_pass1 = _make_sc_scatter(True)
_sc_pass2 = _make_sc_scatter(False)


def kernel(x, edge_index, batch, W1_l, b1_l, W1_r, W2_l, b2_l, W2_r):
    f32 = jnp.float32
    eif = edge_index.astype(jnp.int32).reshape(2 * E)
    xp = x.astype(f32).reshape(NP, 8, D)
    b1p = jnp.tile(b1_l.astype(f32).reshape(1, H), (1, 8))
    b2p = jnp.tile(b2_l.astype(f32).reshape(1, O), (1, 8))
    bt = batch.astype(jnp.int32).reshape(NP, 8).T

    P1p, R1p = _projp(xp, W1_l.astype(f32), W1_r.astype(f32))
    a0, a1, d0, d1 = _sc_pass1(P1p.reshape(N, H), eif)
    P2p, R2bp = _midp(a0.reshape(NP, 128), a1.reshape(NP, 128),
                      d0.reshape(NP, 128), d1.reshape(NP, 128),
                      R1p, b1p, W2_l.astype(f32), W2_r.astype(f32), b2p)
    g0, g1 = _sc_pass2(P2p.reshape(N, O), eif)
    return _finalp(g0.reshape(NP, 128), g1.reshape(NP, 128),
                   d0.reshape(NP, 128), d1.reshape(NP, 128), R2bp, bt)


# speedup vs baseline: 1.0838x; 1.0838x over previous
import functools

import jax
import jax.numpy as jnp
from jax import lax
from jax.experimental import pallas as pl
from jax.experimental.pallas import tpu as pltpu
from jax.experimental.pallas import tpu_sc as plsc

N = 10000
E = 320000
D = 128
H = 16
O = 16
G = 128

NC = 2
NS = 16
NW = NC * NS
EP = E // NW
CH = 1000
NCH = EP // CH
NZ = N // NS
NP = N // 8


def _sc_scatter_body(with_deg, *refs):
    if with_deg:
        (p_hbm, ei_hbm, agg0, agg1, deg0, deg1,
         idx_all, dst_all, rows0, rows1, ones_v, agg_sh, deg_sh,
         isem, gsem, asem, dsem) = refs
    else:
        (p_hbm, ei_hbm, agg0, agg1,
         idx_all, dst_all, rows0, rows1, agg_sh,
         isem, gsem, asem) = refs

    c = lax.axis_index("c")
    s = lax.axis_index("s")
    wid = s * NC + c
    base = wid * EP

    idesc = []
    for i in range(NCH):
        off = base + i * CH
        idesc.append(pltpu.async_copy(
            ei_hbm.at[pl.ds(off, CH)], idx_all.at[i], isem))
        idesc.append(pltpu.async_copy(
            ei_hbm.at[pl.ds(E + off, CH)], dst_all.at[i], isem))

    def _zrow(i, _):
        rows0[i, :] = jnp.zeros((16,), jnp.float32)
        return 0
    lax.fori_loop(0, NZ, _zrow, 0)
    if with_deg:
        def _orow(i, _):
            ones_v[i, :] = jnp.ones((16,), jnp.float32)
            return 0
        lax.fori_loop(0, CH, _orow, 0)

    pltpu.sync_copy(rows0.at[pl.ds(0, NZ)], agg_sh.at[pl.ds(s * NZ, NZ)])
    if with_deg:
        pltpu.sync_copy(rows0.at[pl.ds(0, NZ)], deg_sh.at[pl.ds(s * NZ, NZ)])
    plsc.subcore_barrier()
    for d in idesc:
        d.wait()

    rows = (rows0, rows1)
    g = [None] * NCH
    a = [None] * NCH
    dd = []
    g[0] = pltpu.async_copy(p_hbm.at[idx_all.at[0]], rows[0], gsem)
    for i in range(NCH):
        if i + 1 < NCH:
            if i >= 1:
                a[i - 1].wait()
            g[i + 1] = pltpu.async_copy(
                p_hbm.at[idx_all.at[i + 1]], rows[(i + 1) % 2], gsem)
        g[i].wait()
        a[i] = pltpu.async_copy(
            rows[i % 2], agg_sh.at[dst_all.at[i]], asem, add=True)
        if with_deg:
            dd.append(pltpu.async_copy(
                ones_v, deg_sh.at[dst_all.at[i]], dsem, add=True))
    for i in range(max(0, NCH - 2), NCH):
        a[i].wait()
    for d in dd:
        d.wait()
    plsc.subcore_barrier()

    rb = s * NZ

    @pl.when(c == 0)
    def _():
        pltpu.sync_copy(agg_sh.at[pl.ds(rb, NZ)], agg0.at[pl.ds(rb, NZ)])
        if with_deg:
            pltpu.sync_copy(deg_sh.at[pl.ds(rb, NZ)], deg0.at[pl.ds(rb, NZ)])

    @pl.when(c == 1)
    def _():
        pltpu.sync_copy(agg_sh.at[pl.ds(rb, NZ)], agg1.at[pl.ds(rb, NZ)])
        if with_deg:
            pltpu.sync_copy(deg_sh.at[pl.ds(rb, NZ)], deg1.at[pl.ds(rb, NZ)])


def _make_sc_scatter(with_deg):
    n_out = 4 if with_deg else 2
    scratch = [
        pltpu.VMEM((NCH, CH), jnp.int32),
        pltpu.VMEM((NCH, CH), jnp.int32),
        pltpu.VMEM((CH, 16), jnp.float32),
        pltpu.VMEM((CH, 16), jnp.float32),
    ]
    if with_deg:
        scratch.append(pltpu.VMEM((CH, 16), jnp.float32))
    scratch.append(pltpu.VMEM_SHARED((N, 16), jnp.float32))
    if with_deg:
        scratch.append(pltpu.VMEM_SHARED((N, 16), jnp.float32))
    scratch.extend([pltpu.SemaphoreType.DMA] * (4 if with_deg else 3))

    mesh = plsc.VectorSubcoreMesh(core_axis_name="c", subcore_axis_name="s")
    return functools.partial(
        pl.kernel,
        mesh=mesh,
        out_type=[jax.ShapeDtypeStruct((N, 16), jnp.float32)] * n_out,
        scratch_types=scratch,
        compiler_params=pltpu.CompilerParams(use_tc_tiling_on_sc=False),
    )(functools.partial(_sc_scatter_body, with_deg))


def _sc_deg_body(ei_hbm, deg0, deg1, dst_all, ones_v, zb_v, deg_sh,
                 isem, dsem):
    c = lax.axis_index("c")
    s = lax.axis_index("s")
    wid = s * NC + c
    base = wid * EP

    idesc = []
    for i in range(NCH):
        idesc.append(pltpu.async_copy(
            ei_hbm.at[pl.ds(E + base + i * CH, CH)], dst_all.at[i], isem))

    def _zrow(i, _):
        zb_v[i, :] = jnp.zeros((16,), jnp.float32)
        return 0
    lax.fori_loop(0, NZ, _zrow, 0)

    def _orow(i, _):
        ones_v[i, :] = jnp.ones((16,), jnp.float32)
        return 0
    lax.fori_loop(0, CH, _orow, 0)

    pltpu.sync_copy(zb_v, deg_sh.at[pl.ds(s * NZ, NZ)])
    plsc.subcore_barrier()
    for d in idesc:
        d.wait()

    dd = [pltpu.async_copy(ones_v, deg_sh.at[dst_all.at[i]], dsem, add=True)
          for i in range(NCH)]
    for d in dd:
        d.wait()
    plsc.subcore_barrier()

    rb = s * NZ

    @pl.when(c == 0)
    def _():
        pltpu.sync_copy(deg_sh.at[pl.ds(rb, NZ)], deg0.at[pl.ds(rb, NZ)])

    @pl.when(c == 1)
    def _():
        pltpu.sync_copy(deg_sh.at[pl.ds(rb, NZ)], deg1.at[pl.ds(rb, NZ)])


def _make_sc_deg():
    scratch = [
        pltpu.VMEM((NCH, CH), jnp.int32),
        pltpu.VMEM((CH, 16), jnp.float32),
        pltpu.VMEM((NZ, 16), jnp.float32),
        pltpu.VMEM_SHARED((N, 16), jnp.float32),
        pltpu.SemaphoreType.DMA,
        pltpu.SemaphoreType.DMA,
    ]
    mesh = plsc.VectorSubcoreMesh(core_axis_name="c", subcore_axis_name="s")
    return functools.partial(
        pl.kernel,
        mesh=mesh,
        out_type=[jax.ShapeDtypeStruct((N, 16), jnp.float32)] * 2,
        scratch_types=scratch,
        compiler_params=pltpu.CompilerParams(use_tc_tiling_on_sc=False),
    )(_sc_deg_body)


def _full_spec(shape):
    return pl.BlockSpec(shape, lambda: tuple(0 for _ in shape))


def _dot(a, b):
    return lax.dot_general(a, b, (((1,), (0,)), ((), ())),
                           preferred_element_type=jnp.float32)


def _dot_t(a, b):
    return lax.dot_general(a, b, (((1,), (1,)), ((), ())),
                           preferred_element_type=jnp.float32)


def _blockdiag_matmul(xin, w_t, din):
    outs = [_dot_t(xin[:, din * i:din * (i + 1)], w_t) for i in range(8)]
    return jnp.concatenate(outs, axis=1)


def _projp_body(x3_ref, wl_ref, wr_ref, p_ref, r_ref):
    wl, wr = wl_ref[...], wr_ref[...]
    ps, rs = [], []
    for i in range(8):
        xi = x3_ref[:, i, :]
        ps.append(_dot_t(xi, wl))
        rs.append(_dot_t(xi, wr))
    p_ref[...] = jnp.concatenate(ps, axis=1)
    r_ref[...] = jnp.concatenate(rs, axis=1)


def _midp_body(a0_ref, a1_ref, d0_ref, d1_ref, r1_ref, b1_ref,
               w2l_ref, w2r_ref, b2_ref, p2_ref, r2b_ref):
    deg = jnp.maximum(d0_ref[...] + d1_ref[...], 1.0)
    h = jnp.maximum((a0_ref[...] + a1_ref[...]) / deg
                    + b1_ref[...] + r1_ref[...], 0.0)
    p2_ref[...] = _blockdiag_matmul(h, w2l_ref[...], H)
    r2b_ref[...] = _blockdiag_matmul(h, w2r_ref[...], H) + b2_ref[...]


def _finalp_body(g0_ref, g1_ref, d0_ref, d1_ref, r2b_ref, bt_ref, out_ref):
    deg = jnp.maximum(d0_ref[...] + d1_ref[...], 1.0)
    nodep = (g0_ref[...] + g1_ref[...]) / deg + r2b_ref[...]
    acc = jnp.zeros((G, O), jnp.float32)
    cnt = jnp.zeros((G, 1), jnp.float32)
    for i in range(8):
        bv = bt_ref[i:i + 1, :]
        oh = (lax.broadcasted_iota(jnp.int32, (G, NP), 0)
              == jnp.broadcast_to(bv, (G, NP))).astype(jnp.float32)
        acc += _dot(oh, nodep[:, 16 * i:16 * i + 16])
        cnt += jnp.sum(oh, axis=1, keepdims=True)
    out_ref[...] = acc / jnp.maximum(cnt, 1.0)


def _projp(xp, wl, wr):
    return pl.pallas_call(
        _projp_body,
        in_specs=[_full_spec((NP, 8, D)), _full_spec((H, D)),
                  _full_spec((H, D))],
        out_specs=[_full_spec((NP, 128))] * 2,
        out_shape=[jax.ShapeDtypeStruct((NP, 128), jnp.float32)] * 2,
    )(xp, wl, wr)


def _midp(a0p, a1p, d0p, d1p, r1p, b1p, w2lbd, w2rbd, b2p):
    return pl.pallas_call(
        _midp_body,
        in_specs=[_full_spec((NP, 128))] * 5
        + [_full_spec((1, 128)), _full_spec((O, H)),
           _full_spec((O, H)), _full_spec((1, 128))],
        out_specs=[_full_spec((NP, 128))] * 2,
        out_shape=[jax.ShapeDtypeStruct((NP, 128), jnp.float32)] * 2,
    )(a0p, a1p, d0p, d1p, r1p, b1p, w2lbd, w2rbd, b2p)


def _finalp(g0p, g1p, d0p, d1p, r2bp, bt):
    return pl.pallas_call(
        _finalp_body,
        in_specs=[_full_spec((NP, 128))] * 5 + [_full_spec((8, NP))],
        out_specs=_full_spec((G, O)),
        out_shape=jax.ShapeDtypeStruct((G, O), jnp.float32),
    )(g0p, g1p, d0p, d1p, r2bp, bt)


_sc_pass1 = _make_sc_scatter(True)
_sc_pass2 = _make_sc_scatter(False)


def kernel(x, edge_index, batch, W1_l, b1_l, W1_r, W2_l, b2_l, W2_r):
    f32 = jnp.float32
    eif = edge_index.astype(jnp.int32).reshape(2 * E)
    xp = x.astype(f32).reshape(NP, 8, D)
    b1p = jnp.tile(b1_l.astype(f32).reshape(1, H), (1, 8))
    b2p = jnp.tile(b2_l.astype(f32).reshape(1, O), (1, 8))
    bt = batch.astype(jnp.int32).reshape(NP, 8).T

    P1p, R1p = _projp(xp, W1_l.astype(f32), W1_r.astype(f32))
    a0, a1, d0, d1 = _sc_pass1(P1p.reshape(N, H), eif)
    P2p, R2bp = _midp(a0.reshape(NP, 128), a1.reshape(NP, 128),
                      d0.reshape(NP, 128), d1.reshape(NP, 128),
                      R1p, b1p, W2_l.astype(f32), W2_r.astype(f32), b2p)
    g0, g1 = _sc_pass2(P2p.reshape(N, O), eif)
    return _finalp(g0.reshape(NP, 128), g1.reshape(NP, 128),
                   d0.reshape(NP, 128), d1.reshape(NP, 128), R2bp, bt)


# speedup vs baseline: 1.1657x; 1.0755x over previous
import functools

import jax
import jax.numpy as jnp
from jax import lax
from jax.experimental import pallas as pl
from jax.experimental.pallas import tpu as pltpu
from jax.experimental.pallas import tpu_sc as plsc

N = 10000
E = 320000
D = 128
H = 16
O = 16
G = 128

NC = 2
NS = 16
NW = NC * NS
EP = E // NW
CH = 1000
NCH = EP // CH
NZ = N // NS
NP = N // 8


def _sc_scatter_body(with_deg, *refs):
    if with_deg:
        (p_hbm, ei_hbm, agg0, agg1, deg0, deg1,
         idx_all, dst_all, rows0, rows1, rows2, ones_v, agg_sh, deg_sh,
         isem, gsem, asem, dsem) = refs
    else:
        (p_hbm, ei_hbm, agg0, agg1,
         idx_all, dst_all, rows0, rows1, rows2, agg_sh,
         isem, gsem, asem) = refs

    c = lax.axis_index("c")
    s = lax.axis_index("s")
    wid = s * NC + c
    base = wid * EP

    idesc = []
    for i in range(NCH):
        off = base + i * CH
        idesc.append(pltpu.async_copy(
            ei_hbm.at[pl.ds(off, CH)], idx_all.at[i], isem))
        idesc.append(pltpu.async_copy(
            ei_hbm.at[pl.ds(E + off, CH)], dst_all.at[i], isem))

    def _zrow(i, _):
        rows0[i, :] = jnp.zeros((16,), jnp.float32)
        return 0
    lax.fori_loop(0, NZ, _zrow, 0)
    if with_deg:
        def _orow(i, _):
            ones_v[i, :] = jnp.ones((16,), jnp.float32)
            return 0
        lax.fori_loop(0, CH, _orow, 0)

    pltpu.sync_copy(rows0.at[pl.ds(0, NZ)], agg_sh.at[pl.ds(s * NZ, NZ)])
    if with_deg:
        pltpu.sync_copy(rows0.at[pl.ds(0, NZ)], deg_sh.at[pl.ds(s * NZ, NZ)])
    plsc.subcore_barrier()
    for d in idesc:
        d.wait()

    rows = (rows0, rows1, rows2)
    nb = len(rows)
    g = [None] * NCH
    a = [None] * NCH
    dd = []
    g[0] = pltpu.async_copy(p_hbm.at[idx_all.at[0]], rows[0], gsem)
    for i in range(NCH):
        if i + 1 < NCH:
            if i + 1 >= nb:
                a[i + 1 - nb].wait()
            g[i + 1] = pltpu.async_copy(
                p_hbm.at[idx_all.at[i + 1]], rows[(i + 1) % nb], gsem)
        g[i].wait()
        a[i] = pltpu.async_copy(
            rows[i % nb], agg_sh.at[dst_all.at[i]], asem, add=True)
        if with_deg:
            dd.append(pltpu.async_copy(
                ones_v, deg_sh.at[dst_all.at[i]], dsem, add=True))
    for i in range(max(0, NCH - nb + 1), NCH):
        a[i].wait()
    for d in dd:
        d.wait()
    plsc.subcore_barrier()

    rb = s * NZ

    @pl.when(c == 0)
    def _():
        pltpu.sync_copy(agg_sh.at[pl.ds(rb, NZ)], agg0.at[pl.ds(rb, NZ)])
        if with_deg:
            pltpu.sync_copy(deg_sh.at[pl.ds(rb, NZ)], deg0.at[pl.ds(rb, NZ)])

    @pl.when(c == 1)
    def _():
        pltpu.sync_copy(agg_sh.at[pl.ds(rb, NZ)], agg1.at[pl.ds(rb, NZ)])
        if with_deg:
            pltpu.sync_copy(deg_sh.at[pl.ds(rb, NZ)], deg1.at[pl.ds(rb, NZ)])


def _make_sc_scatter(with_deg):
    n_out = 4 if with_deg else 2
    scratch = [
        pltpu.VMEM((NCH, CH), jnp.int32),
        pltpu.VMEM((NCH, CH), jnp.int32),
        pltpu.VMEM((CH, 16), jnp.float32),
        pltpu.VMEM((CH, 16), jnp.float32),
        pltpu.VMEM((CH, 16), jnp.float32),
    ]
    if with_deg:
        scratch.append(pltpu.VMEM((CH, 16), jnp.float32))
    scratch.append(pltpu.VMEM_SHARED((N, 16), jnp.float32))
    if with_deg:
        scratch.append(pltpu.VMEM_SHARED((N, 16), jnp.float32))
    scratch.extend([pltpu.SemaphoreType.DMA] * (4 if with_deg else 3))

    mesh = plsc.VectorSubcoreMesh(core_axis_name="c", subcore_axis_name="s")
    return functools.partial(
        pl.kernel,
        mesh=mesh,
        out_type=[jax.ShapeDtypeStruct((N, 16), jnp.float32)] * n_out,
        scratch_types=scratch,
        compiler_params=pltpu.CompilerParams(use_tc_tiling_on_sc=False),
    )(functools.partial(_sc_scatter_body, with_deg))


def _full_spec(shape):
    return pl.BlockSpec(shape, lambda: tuple(0 for _ in shape))


def _dot(a, b):
    return lax.dot_general(a, b, (((1,), (0,)), ((), ())),
                           preferred_element_type=jnp.float32)


def _dot_t(a, b):
    return lax.dot_general(a, b, (((1,), (1,)), ((), ())),
                           preferred_element_type=jnp.float32)


def _blockdiag_matmul(xin, w_t, din):
    outs = [_dot_t(xin[:, din * i:din * (i + 1)], w_t) for i in range(8)]
    return jnp.concatenate(outs, axis=1)


def _projp_body(x3_ref, wl_ref, wr_ref, p_ref, r_ref):
    wl, wr = wl_ref[...], wr_ref[...]
    ps, rs = [], []
    for i in range(8):
        xi = x3_ref[:, i, :]
        ps.append(_dot_t(xi, wl))
        rs.append(_dot_t(xi, wr))
    p_ref[...] = jnp.concatenate(ps, axis=1)
    r_ref[...] = jnp.concatenate(rs, axis=1)


def _midp_body(a0_ref, a1_ref, d0_ref, d1_ref, r1_ref, b1_ref,
               w2l_ref, w2r_ref, b2_ref, p2_ref, r2b_ref):
    deg = jnp.maximum(d0_ref[...] + d1_ref[...], 1.0)
    h = jnp.maximum((a0_ref[...] + a1_ref[...]) / deg
                    + b1_ref[...] + r1_ref[...], 0.0)
    p2_ref[...] = _blockdiag_matmul(h, w2l_ref[...], H)
    r2b_ref[...] = _blockdiag_matmul(h, w2r_ref[...], H) + b2_ref[...]


def _finalp_body(g0_ref, g1_ref, d0_ref, d1_ref, r2b_ref, bt_ref, out_ref):
    deg = jnp.maximum(d0_ref[...] + d1_ref[...], 1.0)
    nodep = (g0_ref[...] + g1_ref[...]) / deg + r2b_ref[...]
    acc = jnp.zeros((G, O), jnp.float32)
    cnt = jnp.zeros((G, 1), jnp.float32)
    for i in range(8):
        bv = bt_ref[i:i + 1, :]
        oh = (lax.broadcasted_iota(jnp.int32, (G, NP), 0)
              == jnp.broadcast_to(bv, (G, NP))).astype(jnp.float32)
        acc += _dot(oh, nodep[:, 16 * i:16 * i + 16])
        cnt += jnp.sum(oh, axis=1, keepdims=True)
    out_ref[...] = acc / jnp.maximum(cnt, 1.0)


def _projp(xp, wl, wr):
    return pl.pallas_call(
        _projp_body,
        in_specs=[_full_spec((NP, 8, D)), _full_spec((H, D)),
                  _full_spec((H, D))],
        out_specs=[_full_spec((NP, 128))] * 2,
        out_shape=[jax.ShapeDtypeStruct((NP, 128), jnp.float32)] * 2,
    )(xp, wl, wr)


def _midp(a0p, a1p, d0p, d1p, r1p, b1p, w2lbd, w2rbd, b2p):
    return pl.pallas_call(
        _midp_body,
        in_specs=[_full_spec((NP, 128))] * 5
        + [_full_spec((1, 128)), _full_spec((O, H)),
           _full_spec((O, H)), _full_spec((1, 128))],
        out_specs=[_full_spec((NP, 128))] * 2,
        out_shape=[jax.ShapeDtypeStruct((NP, 128), jnp.float32)] * 2,
    )(a0p, a1p, d0p, d1p, r1p, b1p, w2lbd, w2rbd, b2p)


def _finalp(g0p, g1p, d0p, d1p, r2bp, bt):
    return pl.pallas_call(
        _finalp_body,
        in_specs=[_full_spec((NP, 128))] * 5 + [_full_spec((8, NP))],
        out_specs=_full_spec((G, O)),
        out_shape=jax.ShapeDtypeStruct((G, O), jnp.float32),
    )(g0p, g1p, d0p, d1p, r2bp, bt)


_sc_pass1 = _make_sc_scatter(True)
_sc_pass2 = _make_sc_scatter(False)


def kernel(x, edge_index, batch, W1_l, b1_l, W1_r, W2_l, b2_l, W2_r):
    f32 = jnp.float32
    eif = edge_index.astype(jnp.int32).reshape(2 * E)
    xp = x.astype(f32).reshape(NP, 8, D)
    b1p = jnp.tile(b1_l.astype(f32).reshape(1, H), (1, 8))
    b2p = jnp.tile(b2_l.astype(f32).reshape(1, O), (1, 8))
    bt = batch.astype(jnp.int32).reshape(NP, 8).T

    P1p, R1p = _projp(xp, W1_l.astype(f32), W1_r.astype(f32))
    a0, a1, d0, d1 = _sc_pass1(P1p.reshape(N, H), eif)
    P2p, R2bp = _midp(a0.reshape(NP, 128), a1.reshape(NP, 128),
                      d0.reshape(NP, 128), d1.reshape(NP, 128),
                      R1p, b1p, W2_l.astype(f32), W2_r.astype(f32), b2p)
    g0, g1 = _sc_pass2(P2p.reshape(N, O), eif)
    return _finalp(g0.reshape(NP, 128), g1.reshape(NP, 128),
                   d0.reshape(NP, 128), d1.reshape(NP, 128), R2bp, bt)
